# baseline (speedup 1.0000x reference)
import jax
import jax.numpy as jnp
from jax.experimental import pallas as pl
from jax.experimental.pallas import tpu as pltpu

_NCH = 5
_CH = 2000


def _body(x_hbm, w1_ref, b1_ref, w2_ref, b2_ref, out_hbm,
          x_vm, h_vm, o_vm, insem, outsem):
    for c in range(_NCH):
        pltpu.async_copy(
            x_hbm.at[pl.ds(c * _CH, _CH)],
            x_vm.at[pl.ds(c * _CH, _CH)],
            insem.at[c],
            priority=c % 2,
        )

    w1b = w1_ref[...].astype(jnp.bfloat16)
    w2b = w2_ref[...].astype(jnp.bfloat16)
    b1v = b1_ref[...]
    b2v = b2_ref[...]

    for c in range(_NCH):
        pltpu.make_async_copy(
            x_hbm.at[pl.ds(c * _CH, _CH)],
            x_vm.at[pl.ds(c * _CH, _CH)],
            insem.at[c],
        ).wait()
        xc = x_vm[pl.ds(c * _CH, _CH), :].astype(jnp.bfloat16)
        h = jnp.dot(xc, w1b, preferred_element_type=jnp.float32)
        h_vm[pl.ds(c * _CH, _CH), :] = jnp.maximum(h + b1v, 0.0).astype(
            jnp.bfloat16)

    for c in range(_NCH):
        hc = h_vm[pl.ds(c * _CH, _CH), :]
        o = jnp.dot(hc, w2b, preferred_element_type=jnp.float32) + b2v
        o_vm[pl.ds(c * _CH, _CH), :] = o
        pltpu.async_copy(
            o_vm.at[pl.ds(c * _CH, _CH)],
            out_hbm.at[pl.ds(c * _CH, _CH)],
            outsem.at[c],
            priority=c % 2,
        )

    for c in range(_NCH):
        pltpu.make_async_copy(
            o_vm.at[pl.ds(c * _CH, _CH)],
            out_hbm.at[pl.ds(c * _CH, _CH)],
            outsem.at[c],
        ).wait()


def kernel(x, adj, W1, b1, W2, b2):
    del adj
    n, nfeat = x.shape
    nhid = W1.shape[1]
    nclass = W2.shape[1]
    b1r = b1.reshape(1, nhid)
    b2r = b2.reshape(1, nclass)
    return pl.pallas_call(
        _body,
        in_specs=[
            pl.BlockSpec(memory_space=pltpu.HBM),
            pl.BlockSpec((nfeat, nhid), lambda: (0, 0)),
            pl.BlockSpec((1, nhid), lambda: (0, 0)),
            pl.BlockSpec((nhid, nclass), lambda: (0, 0)),
            pl.BlockSpec((1, nclass), lambda: (0, 0)),
        ],
        out_specs=pl.BlockSpec(memory_space=pltpu.HBM),
        out_shape=jax.ShapeDtypeStruct((n, nclass), jnp.float32),
        scratch_shapes=[
            pltpu.VMEM((n, nfeat), jnp.float32),
            pltpu.VMEM((n, nhid), jnp.bfloat16),
            pltpu.VMEM((n, nclass), jnp.float32),
            pltpu.SemaphoreType.DMA((_NCH,)),
            pltpu.SemaphoreType.DMA((_NCH,)),
        ],
    )(x, W1, b1r, W2, b2r)


# speedup vs baseline: 1.6490x; 1.6490x over previous
import jax
import jax.numpy as jnp
from jax import lax
from jax.experimental import pallas as pl
from jax.experimental.pallas import tpu as pltpu

_NCH = 5
_CH = 2000
_P2 = ((0, 2560), (2560, 2560), (5120, 2560), (7680, 2320))


def _body(x_hbm, w1_ref, b1_ref, w2_ref, b2_ref, out_hbm,
          x_vm, h_vm, o_vm, insem, outsem):
    for c in range(_NCH):
        pltpu.async_copy(
            x_hbm.at[pl.ds(c * _CH, _CH)],
            x_vm.at[pl.ds(c * _CH, _CH)],
            insem.at[c],
            priority=c % 2,
        )

    w1b = w1_ref[...].astype(jnp.bfloat16)
    w2b = w2_ref[...].astype(jnp.bfloat16)
    b1v = b1_ref[...]
    b2c = jnp.reshape(b2_ref[...], (64, 1))

    for c in range(_NCH):
        pltpu.make_async_copy(
            x_hbm.at[pl.ds(c * _CH, _CH)],
            x_vm.at[pl.ds(c * _CH, _CH)],
            insem.at[c],
        ).wait()
        xc = x_vm[pl.ds(c * _CH, _CH), :].astype(jnp.bfloat16)
        h = jnp.dot(xc, w1b, preferred_element_type=jnp.float32)
        h_vm[pl.ds(c * _CH, _CH), :] = jnp.maximum(h + b1v, 0.0).astype(
            jnp.bfloat16)

    for off, sz in _P2:
        hc = h_vm[pl.ds(off, sz), :]
        ot = lax.dot_general(
            w2b, hc, (((0,), (1,)), ((), ())),
            preferred_element_type=jnp.float32)
        o_vm[:, pl.ds(off, sz)] = ot + b2c

    pltpu.async_copy(o_vm, out_hbm, outsem)
    pltpu.make_async_copy(o_vm, out_hbm, outsem).wait()


def kernel(x, adj, W1, b1, W2, b2):
    del adj
    n, nfeat = x.shape
    nhid = W1.shape[1]
    nclass = W2.shape[1]
    b1r = b1.reshape(1, nhid)
    b2r = b2.reshape(1, nclass)
    out_t = pl.pallas_call(
        _body,
        in_specs=[
            pl.BlockSpec(memory_space=pltpu.HBM),
            pl.BlockSpec((nfeat, nhid), lambda: (0, 0)),
            pl.BlockSpec((1, nhid), lambda: (0, 0)),
            pl.BlockSpec((nhid, nclass), lambda: (0, 0)),
            pl.BlockSpec((1, nclass), lambda: (0, 0)),
        ],
        out_specs=pl.BlockSpec(memory_space=pltpu.HBM),
        out_shape=jax.ShapeDtypeStruct((nclass, n), jnp.float32),
        scratch_shapes=[
            pltpu.VMEM((n, nfeat), jnp.float32),
            pltpu.VMEM((n, nhid), jnp.bfloat16),
            pltpu.VMEM((nclass, n), jnp.float32),
            pltpu.SemaphoreType.DMA((_NCH,)),
            pltpu.SemaphoreType.DMA,
        ],
    )(x, W1, b1r, W2, b2r)
    return out_t.T


# speedup vs baseline: 1.7369x; 1.0533x over previous
import jax
import jax.numpy as jnp
from jax import lax
from jax.experimental import pallas as pl
from jax.experimental.pallas import tpu as pltpu

_IN = ((0, 2504), (2504, 2504), (5008, 2504), (7512, 2488))
_P2 = ((0, 4992), (4992, 5008))


def _body(x_hbm, w1_ref, b1_ref, w2_ref, b2_ref, out_hbm,
          x_vm, h_vm, o_vm, dummy_vm, insem, outsem):
    pltpu.touch(dummy_vm)
    for c, (off, sz) in enumerate(_IN):
        pltpu.async_copy(
            x_hbm.at[pl.ds(off, sz)],
            x_vm.at[pl.ds(off, sz)],
            insem.at[c],
            priority=c % 2,
        )

    w1b = w1_ref[...].astype(jnp.bfloat16)
    w2b = w2_ref[...].astype(jnp.bfloat16)
    b1v = b1_ref[...]
    b2c = jnp.reshape(b2_ref[...], (64, 1))

    for c, (off, sz) in enumerate(_IN):
        pltpu.make_async_copy(
            x_hbm.at[pl.ds(off, sz)],
            x_vm.at[pl.ds(off, sz)],
            insem.at[c],
        ).wait()
        xc = x_vm[pl.ds(off, sz), :].astype(jnp.bfloat16)
        h = jnp.dot(xc, w1b, preferred_element_type=jnp.float32)
        h_vm[pl.ds(off, sz), :] = jnp.maximum(h + b1v, 0.0).astype(
            jnp.bfloat16)

    for c, (off, sz) in enumerate(_P2):
        hc = h_vm[pl.ds(off, sz), :]
        ot = lax.dot_general(
            w2b, hc, (((0,), (1,)), ((), ())),
            preferred_element_type=jnp.float32)
        o_vm[:, pl.ds(off, sz)] = ot + b2c
        pltpu.async_copy(
            o_vm.at[:, pl.ds(off, sz)],
            out_hbm.at[:, pl.ds(off, sz)],
            outsem.at[c],
            priority=c % 2,
        )

    for c, (off, sz) in enumerate(_P2):
        pltpu.make_async_copy(
            o_vm.at[:, pl.ds(off, sz)],
            out_hbm.at[:, pl.ds(off, sz)],
            outsem.at[c],
        ).wait()


def kernel(x, adj, W1, b1, W2, b2):
    del adj
    n, nfeat = x.shape
    nhid = W1.shape[1]
    nclass = W2.shape[1]
    b1r = b1.reshape(1, nhid)
    b2r = b2.reshape(1, nclass)
    out_t = pl.pallas_call(
        _body,
        in_specs=[
            pl.BlockSpec(memory_space=pltpu.HBM),
            pl.BlockSpec((nfeat, nhid), lambda: (0, 0)),
            pl.BlockSpec((1, nhid), lambda: (0, 0)),
            pl.BlockSpec((nhid, nclass), lambda: (0, 0)),
            pl.BlockSpec((1, nclass), lambda: (0, 0)),
        ],
        out_specs=pl.BlockSpec(memory_space=pltpu.HBM),
        out_shape=jax.ShapeDtypeStruct((nclass, n), jnp.float32),
        scratch_shapes=[
            pltpu.VMEM((n, nfeat), jnp.float32),
            pltpu.VMEM((n, nhid), jnp.bfloat16),
            pltpu.VMEM((nclass, n), jnp.float32),
            pltpu.VMEM((2048, 2176), jnp.float32),
            pltpu.SemaphoreType.DMA((4,)),
            pltpu.SemaphoreType.DMA((2,)),
        ],
    )(x, W1, b1r, W2, b2r)
    return out_t.T


# speedup vs baseline: 2.2194x; 1.2778x over previous
import jax
import jax.numpy as jnp
from jax import lax
from jax.experimental import pallas as pl
from jax.experimental.pallas import tpu as pltpu

_CHUNKS = tuple((i * 512, 512) for i in range(19)) + ((9728, 272),)
_HALF = ((0, 5120), (5120, 4880))
_HALF_AFTER = {9: 0, 19: 1}


def _body(x_ref, w1_ref, b1_ref, w2_ref, b2_ref, out_hbm, o_vm, outsem):
    w1b = w1_ref[...].astype(jnp.bfloat16)
    w2b = w2_ref[...].astype(jnp.bfloat16)
    b1v = b1_ref[...]
    b2c = jnp.reshape(b2_ref[...], (64, 1))

    for c, (off, sz) in enumerate(_CHUNKS):
        xc = x_ref[pl.ds(off, sz), :].astype(jnp.bfloat16)
        h = jnp.dot(xc, w1b, preferred_element_type=jnp.float32)
        hb = jnp.maximum(h + b1v, 0.0).astype(jnp.bfloat16)
        ot = lax.dot_general(
            w2b, hb, (((0,), (1,)), ((), ())),
            preferred_element_type=jnp.float32)
        o_vm[:, pl.ds(off, sz)] = ot + b2c
        if c in _HALF_AFTER:
            hoff, hsz = _HALF[_HALF_AFTER[c]]
            pltpu.async_copy(
                o_vm.at[:, pl.ds(hoff, hsz)],
                out_hbm.at[:, pl.ds(hoff, hsz)],
                outsem.at[_HALF_AFTER[c]],
                priority=_HALF_AFTER[c] % 2,
            )

    for i, (hoff, hsz) in enumerate(_HALF):
        pltpu.make_async_copy(
            o_vm.at[:, pl.ds(hoff, hsz)],
            out_hbm.at[:, pl.ds(hoff, hsz)],
            outsem.at[i],
        ).wait()


def kernel(x, adj, W1, b1, W2, b2):
    del adj
    n, nfeat = x.shape
    nhid = W1.shape[1]
    nclass = W2.shape[1]
    b1r = b1.reshape(1, nhid)
    b2r = b2.reshape(1, nclass)
    out_t = pl.pallas_call(
        _body,
        in_specs=[
            pl.BlockSpec(memory_space=pltpu.VMEM),
            pl.BlockSpec((nfeat, nhid), lambda: (0, 0)),
            pl.BlockSpec((1, nhid), lambda: (0, 0)),
            pl.BlockSpec((nhid, nclass), lambda: (0, 0)),
            pl.BlockSpec((1, nclass), lambda: (0, 0)),
        ],
        out_specs=pl.BlockSpec(memory_space=pltpu.HBM),
        out_shape=jax.ShapeDtypeStruct((nclass, n), jnp.float32),
        scratch_shapes=[
            pltpu.VMEM((nclass, n), jnp.float32),
            pltpu.SemaphoreType.DMA((2,)),
        ],
    )(x, W1, b1r, W2, b2r)
    return out_t.T
